# Initial kernel scaffold; baseline (speedup 1.0000x reference)
import jax
import jax.numpy as jnp
from jax.experimental import pallas as pl


def kernel(x, edge_index, W_self, W_neigh, b):
    raise NotImplementedError("write your pallas kernel here")


# speedup vs baseline: 4.3853x; 4.3853x over previous
import functools

import jax
import jax.numpy as jnp
from jax import lax
from jax.experimental import pallas as pl
from jax.experimental.pallas import tpu as pltpu
from jax.experimental.pallas import tpu_sc as plsc

NC = 2
NS = 16
NW = NC * NS
LANES = 16


def _zero_f32_2d(ref, nrows, ncols):
    zero = jnp.zeros((LANES,), jnp.float32)
    nc16 = ncols // LANES

    def body(i, carry):
        r = i // nc16
        c = i % nc16
        ref[r, pl.ds(c * LANES, LANES)] = zero
        return carry

    lax.fori_loop(0, nrows * nc16, body, 0)


def _fill_ones_2d(ref, nrows, ncols):
    one = jnp.ones((LANES,), jnp.float32)
    nc16 = ncols // LANES

    def body(i, carry):
        r = i // nc16
        c = i % nc16
        ref[r, pl.ds(c * LANES, LANES)] = one
        return carry

    lax.fori_loop(0, nrows * nc16, body, 0)


def _make_sc_deg(n, e, d):
    C = 80
    epw = e // NW
    nchunk = epw // C
    assert epw * NW == e and nchunk * C == epw
    rpt = (n // (8 * NS)) * 8
    rem = n - NS * rpt
    assert rem % 8 == 0 and rem <= 128
    ZR = rpt // 8
    ncopy = 8

    mesh = plsc.VectorSubcoreMesh(core_axis_name="c", subcore_axis_name="s",
                                  num_cores=NC, num_subcores=NS)

    def body(dst_hbm, deg_out, dst_v, ones_v, zdeg, deg_sh, sem):
        del sem
        cid = lax.axis_index("c")
        sid = lax.axis_index("s")
        wid = sid * NC + cid
        r0 = sid * rpt

        _zero_f32_2d(zdeg, ZR, d)
        _fill_ones_2d(ones_v, C, d)

        def zdbody(j, carry):
            pltpu.sync_copy(zdeg, deg_sh.at[pl.ds(r0 + j * ZR, ZR)])
            return carry

        lax.fori_loop(0, ncopy, zdbody, 0)
        if rem:
            @pl.when(sid == NS - 1)
            def _():
                pltpu.sync_copy(zdeg.at[pl.ds(0, rem)],
                                deg_sh.at[pl.ds(NS * rpt, rem)])
        plsc.subcore_barrier()

        base_w = wid * epw

        def chunk(i, carry):
            base = base_w + i * C
            pltpu.sync_copy(dst_hbm.at[pl.ds(base, C)], dst_v)
            pltpu.sync_copy(ones_v, deg_sh.at[dst_v], add=True)
            return carry

        lax.fori_loop(0, nchunk, chunk, 0)
        plsc.subcore_barrier()

        pltpu.sync_copy(deg_sh.at[pl.ds(r0, rpt)],
                        deg_out.at[cid, pl.ds(r0, rpt)])
        if rem:
            @pl.when(sid == NS - 1)
            def _():
                pltpu.sync_copy(deg_sh.at[pl.ds(NS * rpt, rem)],
                                deg_out.at[cid, pl.ds(NS * rpt, rem)])

    return pl.kernel(
        body,
        out_type=[jax.ShapeDtypeStruct((NC, n, d), jnp.float32)],
        mesh=mesh,
        scratch_types=[
            pltpu.VMEM((C,), jnp.int32),
            pltpu.VMEM((C, d), jnp.float32),
            pltpu.VMEM((ZR, d), jnp.float32),
            pltpu.VMEM_SHARED((n, d), jnp.float32),
            pltpu.SemaphoreType.DMA,
        ])


def _make_sc_agg(n, e, d):
    C = 80
    epw = e // NW
    nchunk = epw // C
    assert epw * NW == e and nchunk * C == epw
    rpt = (n // (8 * NS)) * 8
    rem = n - NS * rpt
    assert rem % 8 == 0 and rem <= 128
    ZR = rpt // 8
    ncopy = 8

    mesh = plsc.VectorSubcoreMesh(core_axis_name="c", subcore_axis_name="s",
                                  num_cores=NC, num_subcores=NS)

    out_type = [jax.ShapeDtypeStruct((NC, n, d), jnp.float32)]
    scratch = [
        pltpu.VMEM((C,), jnp.int32),
        pltpu.VMEM((C,), jnp.int32),
        pltpu.VMEM((C, d), jnp.float32),
        pltpu.VMEM((ZR, d), jnp.float32),
        pltpu.VMEM_SHARED((n, d), jnp.float32),
        pltpu.SemaphoreType.DMA,
    ]

    def body(h_hbm, src_hbm, dst_hbm, agg_out, src_v, dst_v, rows_v, zrow,
             acc_sh, sem):
        cid = lax.axis_index("c")
        sid = lax.axis_index("s")
        wid = sid * NC + cid

        _zero_f32_2d(zrow, ZR, d)
        r0 = sid * rpt

        def zbody(j, carry):
            pltpu.sync_copy(zrow, acc_sh.at[pl.ds(r0 + j * ZR, ZR)])
            return carry

        lax.fori_loop(0, ncopy, zbody, 0)
        if rem:
            @pl.when(sid == NS - 1)
            def _():
                pltpu.sync_copy(zrow.at[pl.ds(0, rem)],
                                acc_sh.at[pl.ds(NS * rpt, rem)])
        plsc.subcore_barrier()

        base_w = wid * epw

        def chunk(i, carry):
            base = base_w + i * C
            pltpu.sync_copy(src_hbm.at[pl.ds(base, C)], src_v)
            pltpu.sync_copy(dst_hbm.at[pl.ds(base, C)], dst_v)
            pltpu.async_copy(h_hbm.at[src_v], rows_v, sem).wait()
            pltpu.sync_copy(rows_v, acc_sh.at[dst_v], add=True)
            return carry

        lax.fori_loop(0, nchunk, chunk, 0)
        plsc.subcore_barrier()

        pltpu.sync_copy(acc_sh.at[pl.ds(r0, rpt)],
                        agg_out.at[cid, pl.ds(r0, rpt)])
        if rem:
            @pl.when(sid == NS - 1)
            def _():
                pltpu.sync_copy(acc_sh.at[pl.ds(NS * rpt, rem)],
                                agg_out.at[cid, pl.ds(NS * rpt, rem)])

    return pl.kernel(body, out_type=out_type, mesh=mesh,
                     scratch_types=scratch)


def _make_tc_layer(n, d, relu):
    R = 2000
    assert n % R == 0

    def body(h_ref, ap_ref, dp_ref, ws_ref, wn_ref, b_ref, o_ref):
        deg = jnp.maximum(dp_ref[0, :, 0] + dp_ref[1, :, 0], 1.0)
        agg = (ap_ref[0] + ap_ref[1]) / deg[:, None]
        acc = jnp.dot(h_ref[...], ws_ref[...],
                      preferred_element_type=jnp.float32)
        acc = acc + jnp.dot(agg, wn_ref[...],
                            preferred_element_type=jnp.float32)
        acc = acc + b_ref[...]
        o_ref[...] = jnp.maximum(acc, 0.0) if relu else acc

    return pl.pallas_call(
        body,
        grid=(n // R,),
        in_specs=[
            pl.BlockSpec((R, d), lambda i: (i, 0)),
            pl.BlockSpec((NC, R, d), lambda i: (0, i, 0)),
            pl.BlockSpec((NC, R, d), lambda i: (0, i, 0)),
            pl.BlockSpec((d, d), lambda i: (0, 0)),
            pl.BlockSpec((d, d), lambda i: (0, 0)),
            pl.BlockSpec((1, d), lambda i: (0, 0)),
        ],
        out_specs=pl.BlockSpec((R, d), lambda i: (i, 0)),
        out_shape=jax.ShapeDtypeStruct((n, d), jnp.float32),
    )


@functools.lru_cache(maxsize=None)
def _make_pipeline(n, e, d):
    sc_deg = _make_sc_deg(n, e, d)
    sc_agg = _make_sc_agg(n, e, d)
    tc_relu = _make_tc_layer(n, d, relu=True)
    tc_lin = _make_tc_layer(n, d, relu=False)
    return sc_deg, sc_agg, tc_relu, tc_lin


def kernel(x, edge_index, W_self, W_neigh, b):
    n, d = x.shape
    e = edge_index.shape[1]
    sc_deg, sc_agg, tc_relu, tc_lin = _make_pipeline(n, e, d)

    src = edge_index[0]
    dst = edge_index[1]
    wst = W_self.T
    wnt = W_neigh.T
    b2 = b[None, :]

    (degp,) = sc_deg(dst)
    (aggp,) = sc_agg(x, src, dst)
    h = tc_relu(x, aggp, degp, wst, wnt, b2)
    (aggp,) = sc_agg(h, src, dst)
    h = tc_relu(h, aggp, degp, wst, wnt, b2)
    (aggp,) = sc_agg(h, src, dst)
    return tc_lin(h, aggp, degp, wst, wnt, b2)


# speedup vs baseline: 9.0181x; 2.0564x over previous
import functools

import jax
import jax.numpy as jnp
from jax import lax
from jax.experimental import pallas as pl
from jax.experimental.pallas import tpu as pltpu
from jax.experimental.pallas import tpu_sc as plsc

NC = 2
NS = 16
NW = NC * NS
LANES = 16
CHUNK = 80


def _zero_f32_2d(ref, nrows, ncols):
    zero = jnp.zeros((LANES,), jnp.float32)
    nc16 = ncols // LANES

    def body(i, carry):
        r = i // nc16
        c = i % nc16
        ref[r, pl.ds(c * LANES, LANES)] = zero
        return carry

    lax.fori_loop(0, nrows * nc16, body, 0)


def _fill_ones_2d(ref, nrows, ncols):
    one = jnp.ones((LANES,), jnp.float32)
    nc16 = ncols // LANES

    def body(i, carry):
        r = i // nc16
        c = i % nc16
        ref[r, pl.ds(c * LANES, LANES)] = one
        return carry

    lax.fori_loop(0, nrows * nc16, body, 0)


def _make_sc_deg(n, e, d):
    C = 80
    epw = e // NW
    nchunk = epw // C
    assert epw * NW == e and nchunk * C == epw
    rpt = (n // (8 * NS)) * 8
    rem = n - NS * rpt
    assert rem % 8 == 0 and rem <= 128
    ZR = rpt // 8
    ncopy = 8

    mesh = plsc.VectorSubcoreMesh(core_axis_name="c", subcore_axis_name="s",
                                  num_cores=NC, num_subcores=NS)

    def body(dst_hbm, deg_out, dst_v, ones_v, zdeg, deg_sh, sem):
        del sem
        cid = lax.axis_index("c")
        sid = lax.axis_index("s")
        wid = sid * NC + cid
        r0 = sid * rpt

        _zero_f32_2d(zdeg, ZR, d)
        _fill_ones_2d(ones_v, C, d)

        def zdbody(j, carry):
            pltpu.sync_copy(zdeg, deg_sh.at[pl.ds(r0 + j * ZR, ZR)])
            return carry

        lax.fori_loop(0, ncopy, zdbody, 0)
        if rem:
            @pl.when(sid == NS - 1)
            def _():
                pltpu.sync_copy(zdeg.at[pl.ds(0, rem)],
                                deg_sh.at[pl.ds(NS * rpt, rem)])
        plsc.subcore_barrier()

        base_w = wid * epw

        def chunk(i, carry):
            base = base_w + i * C
            pltpu.sync_copy(dst_hbm.at[pl.ds(base, C)], dst_v)
            pltpu.sync_copy(ones_v, deg_sh.at[dst_v], add=True)
            return carry

        lax.fori_loop(0, nchunk, chunk, 0)
        plsc.subcore_barrier()

        pltpu.sync_copy(deg_sh.at[pl.ds(r0, rpt)],
                        deg_out.at[cid, pl.ds(r0, rpt)])
        if rem:
            @pl.when(sid == NS - 1)
            def _():
                pltpu.sync_copy(deg_sh.at[pl.ds(NS * rpt, rem)],
                                deg_out.at[cid, pl.ds(NS * rpt, rem)])

    return pl.kernel(
        body,
        out_type=[jax.ShapeDtypeStruct((NC, n, d), jnp.float32)],
        mesh=mesh,
        scratch_types=[
            pltpu.VMEM((C,), jnp.int32),
            pltpu.VMEM((C, d), jnp.float32),
            pltpu.VMEM((ZR, d), jnp.float32),
            pltpu.VMEM_SHARED((n, d), jnp.float32),
            pltpu.SemaphoreType.DMA,
        ])


def _make_sc_agg(n, e, d):
    C = CHUNK
    epw = e // NW
    nchunk = epw // C
    assert epw * NW == e and nchunk * C == epw
    rpt = (n // (8 * NS)) * 8
    rem = n - NS * rpt
    assert rem % 8 == 0 and rem <= 128
    ZR = rpt // 8
    ncopy = 8

    mesh = plsc.VectorSubcoreMesh(core_axis_name="c", subcore_axis_name="s",
                                  num_cores=NC, num_subcores=NS)

    assert nchunk % 2 == 1

    out_type = [jax.ShapeDtypeStruct((NC, n, d), jnp.float32)]
    scratch = [
        pltpu.VMEM((epw,), jnp.int32),
        pltpu.VMEM((C,), jnp.int32),
        pltpu.VMEM((C,), jnp.int32),
        pltpu.VMEM((C, d), jnp.float32),
        pltpu.VMEM((C, d), jnp.float32),
        pltpu.VMEM((ZR, d), jnp.float32),
        pltpu.VMEM_SHARED((n, d), jnp.float32),
        pltpu.SemaphoreType.DMA,
        pltpu.SemaphoreType.DMA,
        pltpu.SemaphoreType.DMA,
        pltpu.SemaphoreType.DMA,
    ]

    def body(h_hbm, src_hbm, dst_hbm, agg_out, src_v, dstb0, dstb1, rows0,
             rows1, zrow, acc_sh, sem0, sem1, dsem0, dsem1):
        cid = lax.axis_index("c")
        sid = lax.axis_index("s")
        wid = sid * NC + cid
        base_w = wid * epw

        pltpu.sync_copy(src_hbm.at[pl.ds(base_w, epw)], src_v)
        _zero_f32_2d(zrow, ZR, d)
        r0 = sid * rpt

        def zbody(j, carry):
            pltpu.sync_copy(zrow, acc_sh.at[pl.ds(r0 + j * ZR, ZR)])
            return carry

        lax.fori_loop(0, ncopy, zbody, 0)
        if rem:
            @pl.when(sid == NS - 1)
            def _():
                pltpu.sync_copy(zrow.at[pl.ds(0, rem)],
                                acc_sh.at[pl.ds(NS * rpt, rem)])
        plsc.subcore_barrier()

        def start_gather(i, rows, dstb, sem, dsem):
            pltpu.async_copy(h_hbm.at[src_v.at[pl.ds(i * C, C)]], rows, sem)
            pltpu.async_copy(dst_hbm.at[pl.ds(base_w + i * C, C)], dstb,
                             dsem)

        def finish(i, rows, dstb, sem, dsem):
            del i
            pltpu.make_async_copy(h_hbm.at[pl.ds(0, C)], rows, sem).wait()
            pltpu.make_async_copy(dst_hbm.at[pl.ds(0, C)], dstb, dsem).wait()
            pltpu.sync_copy(rows, acc_sh.at[dstb], add=True)

        start_gather(0, rows0, dstb0, sem0, dsem0)
        start_gather(1, rows1, dstb1, sem1, dsem1)

        def pair(j, carry):
            i = 2 * j
            finish(i, rows0, dstb0, sem0, dsem0)

            @pl.when(i + 2 < nchunk)
            def _():
                start_gather(i + 2, rows0, dstb0, sem0, dsem0)
            finish(i + 1, rows1, dstb1, sem1, dsem1)

            @pl.when(i + 3 < nchunk)
            def _():
                start_gather(i + 3, rows1, dstb1, sem1, dsem1)
            return carry

        lax.fori_loop(0, nchunk // 2, pair, 0)
        finish(nchunk - 1, rows0, dstb0, sem0, dsem0)
        plsc.subcore_barrier()

        pltpu.sync_copy(acc_sh.at[pl.ds(r0, rpt)],
                        agg_out.at[cid, pl.ds(r0, rpt)])
        if rem:
            @pl.when(sid == NS - 1)
            def _():
                pltpu.sync_copy(acc_sh.at[pl.ds(NS * rpt, rem)],
                                agg_out.at[cid, pl.ds(NS * rpt, rem)])

    return pl.kernel(body, out_type=out_type, mesh=mesh,
                     scratch_types=scratch)


def _make_tc_layer(n, d, relu):
    R = 2000
    assert n % R == 0

    def body(h_ref, ap_ref, dp_ref, ws_ref, wn_ref, b_ref, o_ref):
        deg = jnp.maximum(dp_ref[0, :, 0] + dp_ref[1, :, 0], 1.0)
        agg = (ap_ref[0] + ap_ref[1]) / deg[:, None]
        acc = jnp.dot(h_ref[...], ws_ref[...],
                      preferred_element_type=jnp.float32)
        acc = acc + jnp.dot(agg, wn_ref[...],
                            preferred_element_type=jnp.float32)
        acc = acc + b_ref[...]
        o_ref[...] = jnp.maximum(acc, 0.0) if relu else acc

    return pl.pallas_call(
        body,
        grid=(n // R,),
        in_specs=[
            pl.BlockSpec((R, d), lambda i: (i, 0)),
            pl.BlockSpec((NC, R, d), lambda i: (0, i, 0)),
            pl.BlockSpec((NC, R, d), lambda i: (0, i, 0)),
            pl.BlockSpec((d, d), lambda i: (0, 0)),
            pl.BlockSpec((d, d), lambda i: (0, 0)),
            pl.BlockSpec((1, d), lambda i: (0, 0)),
        ],
        out_specs=pl.BlockSpec((R, d), lambda i: (i, 0)),
        out_shape=jax.ShapeDtypeStruct((n, d), jnp.float32),
    )


@functools.lru_cache(maxsize=None)
def _make_pipeline(n, e, d):
    sc_deg = _make_sc_deg(n, e, d)
    sc_agg = _make_sc_agg(n, e, d)
    tc_relu = _make_tc_layer(n, d, relu=True)
    tc_lin = _make_tc_layer(n, d, relu=False)
    return sc_deg, sc_agg, tc_relu, tc_lin


def kernel(x, edge_index, W_self, W_neigh, b):
    n, d = x.shape
    e = edge_index.shape[1]
    sc_deg, sc_agg, tc_relu, tc_lin = _make_pipeline(n, e, d)

    src = edge_index[0]
    dst = edge_index[1]
    wst = W_self.T
    wnt = W_neigh.T
    b2 = b[None, :]

    (degp,) = sc_deg(dst)
    (aggp,) = sc_agg(x, src, dst)
    h = tc_relu(x, aggp, degp, wst, wnt, b2)
    (aggp,) = sc_agg(h, src, dst)
    h = tc_relu(h, aggp, degp, wst, wnt, b2)
    (aggp,) = sc_agg(h, src, dst)
    return tc_lin(h, aggp, degp, wst, wnt, b2)


# speedup vs baseline: 9.9644x; 1.1049x over previous
import functools

import jax
import jax.numpy as jnp
from jax import lax
from jax.experimental import pallas as pl
from jax.experimental.pallas import tpu as pltpu
from jax.experimental.pallas import tpu_sc as plsc

NC = 2
NS = 16
NW = NC * NS
LANES = 16
CHUNK = 80


def _zero_f32_2d(ref, nrows, ncols):
    zero = jnp.zeros((LANES,), jnp.float32)
    nc16 = ncols // LANES

    def body(i, carry):
        r = i // nc16
        c = i % nc16
        ref[r, pl.ds(c * LANES, LANES)] = zero
        return carry

    lax.fori_loop(0, nrows * nc16, body, 0)


def _fill_ones_2d(ref, nrows, ncols):
    one = jnp.ones((LANES,), jnp.float32)
    nc16 = ncols // LANES

    def body(i, carry):
        r = i // nc16
        c = i % nc16
        ref[r, pl.ds(c * LANES, LANES)] = one
        return carry

    lax.fori_loop(0, nrows * nc16, body, 0)


def _make_sc_deg(n, e, d):
    C = 80
    epw = e // NW
    nchunk = epw // C
    assert epw * NW == e and nchunk * C == epw
    rpt = (n // (8 * NS)) * 8
    rem = n - NS * rpt
    assert rem % 8 == 0 and rem <= 128
    ZR = rpt // 8
    ncopy = 8

    mesh = plsc.VectorSubcoreMesh(core_axis_name="c", subcore_axis_name="s",
                                  num_cores=NC, num_subcores=NS)

    assert nchunk % 2 == 1

    def body(dst_hbm, deg_out, dstb0, dstb1, ones_v, zdeg, deg_sh, dsem0,
             dsem1):
        cid = lax.axis_index("c")
        sid = lax.axis_index("s")
        wid = sid * NC + cid
        r0 = sid * rpt
        base_w = wid * epw

        _zero_f32_2d(zdeg, ZR, d)
        _fill_ones_2d(ones_v, C, d)

        def zdbody(j, carry):
            pltpu.sync_copy(zdeg, deg_sh.at[pl.ds(r0 + j * ZR, ZR)])
            return carry

        lax.fori_loop(0, ncopy, zdbody, 0)
        if rem:
            @pl.when(sid == NS - 1)
            def _():
                pltpu.sync_copy(zdeg.at[pl.ds(0, rem)],
                                deg_sh.at[pl.ds(NS * rpt, rem)])
        plsc.subcore_barrier()

        def start(i, dstb, dsem):
            pltpu.async_copy(dst_hbm.at[pl.ds(base_w + i * C, C)], dstb,
                             dsem)

        def finish(dstb, dsem):
            pltpu.make_async_copy(dst_hbm.at[pl.ds(0, C)], dstb,
                                  dsem).wait()
            pltpu.sync_copy(ones_v, deg_sh.at[dstb], add=True)

        start(0, dstb0, dsem0)
        start(1, dstb1, dsem1)

        def pair(j, carry):
            i = 2 * j
            finish(dstb0, dsem0)

            @pl.when(i + 2 < nchunk)
            def _():
                start(i + 2, dstb0, dsem0)
            finish(dstb1, dsem1)

            @pl.when(i + 3 < nchunk)
            def _():
                start(i + 3, dstb1, dsem1)
            return carry

        lax.fori_loop(0, nchunk // 2, pair, 0)
        finish(dstb0, dsem0)
        plsc.subcore_barrier()

        pltpu.sync_copy(deg_sh.at[pl.ds(r0, rpt)],
                        deg_out.at[cid, pl.ds(r0, rpt)])
        if rem:
            @pl.when(sid == NS - 1)
            def _():
                pltpu.sync_copy(deg_sh.at[pl.ds(NS * rpt, rem)],
                                deg_out.at[cid, pl.ds(NS * rpt, rem)])

    return pl.kernel(
        body,
        out_type=[jax.ShapeDtypeStruct((NC, n, d), jnp.float32)],
        mesh=mesh,
        scratch_types=[
            pltpu.VMEM((C,), jnp.int32),
            pltpu.VMEM((C,), jnp.int32),
            pltpu.VMEM((C, d), jnp.float32),
            pltpu.VMEM((ZR, d), jnp.float32),
            pltpu.VMEM_SHARED((n, d), jnp.float32),
            pltpu.SemaphoreType.DMA,
            pltpu.SemaphoreType.DMA,
        ])


def _make_sc_agg(n, e, d):
    C = CHUNK
    epw = e // NW
    nchunk = epw // C
    assert epw * NW == e and nchunk * C == epw
    rpt = (n // (8 * NS)) * 8
    rem = n - NS * rpt
    assert rem % 8 == 0 and rem <= 128
    ZR = rpt // 8
    ncopy = 8

    mesh = plsc.VectorSubcoreMesh(core_axis_name="c", subcore_axis_name="s",
                                  num_cores=NC, num_subcores=NS)

    assert nchunk % 2 == 1

    out_type = [jax.ShapeDtypeStruct((NC, n, d), jnp.float32)]
    scratch = [
        pltpu.VMEM((epw,), jnp.int32),
        pltpu.VMEM((C,), jnp.int32),
        pltpu.VMEM((C,), jnp.int32),
        pltpu.VMEM((C, d), jnp.float32),
        pltpu.VMEM((C, d), jnp.float32),
        pltpu.VMEM((ZR, d), jnp.float32),
        pltpu.VMEM_SHARED((n, d), jnp.float32),
        pltpu.SemaphoreType.DMA,
        pltpu.SemaphoreType.DMA,
        pltpu.SemaphoreType.DMA,
        pltpu.SemaphoreType.DMA,
    ]

    def body(h_hbm, src_hbm, dst_hbm, agg_out, src_v, dstb0, dstb1, rows0,
             rows1, zrow, acc_sh, sem0, sem1, dsem0, dsem1):
        cid = lax.axis_index("c")
        sid = lax.axis_index("s")
        wid = sid * NC + cid
        base_w = wid * epw

        pltpu.sync_copy(src_hbm.at[pl.ds(base_w, epw)], src_v)
        _zero_f32_2d(zrow, ZR, d)
        r0 = sid * rpt

        def zbody(j, carry):
            pltpu.sync_copy(zrow, acc_sh.at[pl.ds(r0 + j * ZR, ZR)])
            return carry

        lax.fori_loop(0, ncopy, zbody, 0)
        if rem:
            @pl.when(sid == NS - 1)
            def _():
                pltpu.sync_copy(zrow.at[pl.ds(0, rem)],
                                acc_sh.at[pl.ds(NS * rpt, rem)])
        plsc.subcore_barrier()

        def start_gather(i, rows, dstb, sem, dsem):
            pltpu.async_copy(h_hbm.at[src_v.at[pl.ds(i * C, C)]], rows, sem)
            pltpu.async_copy(dst_hbm.at[pl.ds(base_w + i * C, C)], dstb,
                             dsem)

        def finish(i, rows, dstb, sem, dsem):
            del i
            pltpu.make_async_copy(h_hbm.at[pl.ds(0, C)], rows, sem).wait()
            pltpu.make_async_copy(dst_hbm.at[pl.ds(0, C)], dstb, dsem).wait()
            pltpu.sync_copy(rows, acc_sh.at[dstb], add=True)

        start_gather(0, rows0, dstb0, sem0, dsem0)
        start_gather(1, rows1, dstb1, sem1, dsem1)

        def pair(j, carry):
            i = 2 * j
            finish(i, rows0, dstb0, sem0, dsem0)

            @pl.when(i + 2 < nchunk)
            def _():
                start_gather(i + 2, rows0, dstb0, sem0, dsem0)
            finish(i + 1, rows1, dstb1, sem1, dsem1)

            @pl.when(i + 3 < nchunk)
            def _():
                start_gather(i + 3, rows1, dstb1, sem1, dsem1)
            return carry

        lax.fori_loop(0, nchunk // 2, pair, 0)
        finish(nchunk - 1, rows0, dstb0, sem0, dsem0)
        plsc.subcore_barrier()

        pltpu.sync_copy(acc_sh.at[pl.ds(r0, rpt)],
                        agg_out.at[cid, pl.ds(r0, rpt)])
        if rem:
            @pl.when(sid == NS - 1)
            def _():
                pltpu.sync_copy(acc_sh.at[pl.ds(NS * rpt, rem)],
                                agg_out.at[cid, pl.ds(NS * rpt, rem)])

    return pl.kernel(body, out_type=out_type, mesh=mesh,
                     scratch_types=scratch)


def _make_tc_layer1(n, d):
    R = 2000
    assert n % R == 0

    def body(h_ref, ap_ref, dp_ref, ws_ref, wn_ref, b_ref, o_ref, rd_ref):
        deg = jnp.maximum(dp_ref[0, :, 0] + dp_ref[1, :, 0], 1.0)
        rd = (1.0 / deg)[:, None]
        rd_ref[...] = rd
        agg = (ap_ref[0] + ap_ref[1]) * rd
        acc = jnp.dot(h_ref[...], ws_ref[...],
                      preferred_element_type=jnp.float32)
        acc = acc + jnp.dot(agg, wn_ref[...],
                            preferred_element_type=jnp.float32)
        o_ref[...] = jnp.maximum(acc + b_ref[...], 0.0)

    return pl.pallas_call(
        body,
        grid=(n // R,),
        in_specs=[
            pl.BlockSpec((R, d), lambda i: (i, 0)),
            pl.BlockSpec((NC, R, d), lambda i: (0, i, 0)),
            pl.BlockSpec((NC, R, d), lambda i: (0, i, 0)),
            pl.BlockSpec((d, d), lambda i: (0, 0)),
            pl.BlockSpec((d, d), lambda i: (0, 0)),
            pl.BlockSpec((1, d), lambda i: (0, 0)),
        ],
        out_specs=[
            pl.BlockSpec((R, d), lambda i: (i, 0)),
            pl.BlockSpec((R, 1), lambda i: (i, 0)),
        ],
        out_shape=[
            jax.ShapeDtypeStruct((n, d), jnp.float32),
            jax.ShapeDtypeStruct((n, 1), jnp.float32),
        ],
    )


def _make_tc_layer(n, d, relu):
    R = 2000
    assert n % R == 0

    def body(h_ref, ap_ref, rd_ref, ws_ref, wn_ref, b_ref, o_ref):
        agg = (ap_ref[0] + ap_ref[1]) * rd_ref[...]
        acc = jnp.dot(h_ref[...], ws_ref[...],
                      preferred_element_type=jnp.float32)
        acc = acc + jnp.dot(agg, wn_ref[...],
                            preferred_element_type=jnp.float32)
        acc = acc + b_ref[...]
        o_ref[...] = jnp.maximum(acc, 0.0) if relu else acc

    return pl.pallas_call(
        body,
        grid=(n // R,),
        in_specs=[
            pl.BlockSpec((R, d), lambda i: (i, 0)),
            pl.BlockSpec((NC, R, d), lambda i: (0, i, 0)),
            pl.BlockSpec((R, 1), lambda i: (i, 0)),
            pl.BlockSpec((d, d), lambda i: (0, 0)),
            pl.BlockSpec((d, d), lambda i: (0, 0)),
            pl.BlockSpec((1, d), lambda i: (0, 0)),
        ],
        out_specs=pl.BlockSpec((R, d), lambda i: (i, 0)),
        out_shape=jax.ShapeDtypeStruct((n, d), jnp.float32),
    )


@functools.lru_cache(maxsize=None)
def _make_pipeline(n, e, d):
    sc_deg = _make_sc_deg(n, e, d)
    sc_agg = _make_sc_agg(n, e, d)
    tc_first = _make_tc_layer1(n, d)
    tc_relu = _make_tc_layer(n, d, relu=True)
    tc_lin = _make_tc_layer(n, d, relu=False)
    return sc_deg, sc_agg, tc_first, tc_relu, tc_lin


def kernel(x, edge_index, W_self, W_neigh, b):
    n, d = x.shape
    e = edge_index.shape[1]
    sc_deg, sc_agg, tc_first, tc_relu, tc_lin = _make_pipeline(n, e, d)

    src = edge_index[0]
    dst = edge_index[1]
    wst = W_self.T
    wnt = W_neigh.T
    b2 = b[None, :]

    (degp,) = sc_deg(dst)
    (aggp,) = sc_agg(x, src, dst)
    h, rdeg = tc_first(x, aggp, degp, wst, wnt, b2)
    (aggp,) = sc_agg(h, src, dst)
    h = tc_relu(h, aggp, rdeg, wst, wnt, b2)
    (aggp,) = sc_agg(h, src, dst)
    return tc_lin(h, aggp, rdeg, wst, wnt, b2)
